# baseline (speedup 1.0000x reference)
import functools

import jax
import jax.numpy as jnp
from jax import lax
from jax.experimental import pallas as pl
from jax.experimental.pallas import tpu as pltpu
from jax.experimental.pallas import tpu_sc as plsc

EPS = 1e-5

NC = 2
NS = 16
NW = NC * NS

KC = 304
WT = 10


def _make_spmm(N, E, S, with_deg=False, interpret=False):
    EPT = E // NW
    FULL = EPT // KC
    TAILB = FULL * KC
    TAILN = EPT - TAILB
    CH = FULL + 1
    assert TAILN % 16 == 8
    RPT = N // WT
    mesh = plsc.VectorSubcoreMesh(core_axis_name="c", subcore_axis_name="s")

    parts_t = jax.ShapeDtypeStruct((NC * S * N, 128), jnp.float32)
    out_t = [parts_t, jax.ShapeDtypeStruct((NC * N, 128), jnp.float32)] \
        if with_deg else parts_t

    @functools.partial(
        pl.kernel,
        out_type=out_t,
        mesh=mesh,
        interpret=interpret,
        scratch_types=[
            pltpu.VMEM((CH * KC,), jnp.int32),
            pltpu.VMEM((EPT + 16,), jnp.int32),
            pltpu.VMEM((KC,), jnp.int32),
            pltpu.VMEM((KC, 128), jnp.float32),
            pltpu.VMEM_SHARED((N + 8, 128), jnp.float32),
            pltpu.SemaphoreType.DMA,
            pltpu.SemaphoreType.DMA,
        ],
    )
    def spmm(*refs):
        tabs = refs[:S]
        src, dst, zeros, ones = refs[S:S + 4]
        if with_deg:
            (out, dout, src_all, dst_all, sidx, rows, acc,
             semG, semS) = refs[S + 4:]
        else:
            (out, src_all, dst_all, sidx, rows, acc,
             semG, semS) = refs[S + 4:]
        cid = lax.axis_index("c")
        sid = lax.axis_index("s")
        wid = cid * NS + sid
        ebase = wid * EPT
        NB = KC * 128 * 4

        pltpu.sync_copy(src.at[pl.ds(ebase, EPT)], src_all.at[pl.ds(0, EPT)])
        pltpu.sync_copy(dst.at[pl.ds(ebase, EPT)], dst_all.at[pl.ds(0, EPT)])
        lane = lax.iota(jnp.int32, 16)
        nf = (TAILN // 16) * 16
        sv = src_all[pl.ds(TAILB + nf, 16)]
        src_all[pl.ds(TAILB + nf, 16)] = jnp.where(lane < TAILN - nf, sv, 0)
        for o in range(nf + 16, KC, 16):
            src_all[pl.ds(TAILB + o, 16)] = jnp.zeros((16,), jnp.int32)

        def build_s(g, sidx):
            for i in range(KC // 16):
                sidx[pl.ds(i * 16, 16)] = dst_all[pl.ds(g * KC + i * 16, 16)]

        def build_s_tail(sidx):
            for i in range(TAILN // 16):
                sidx[pl.ds(i * 16, 16)] = \
                    dst_all[pl.ds(TAILB + i * 16, 16)]
            dv = dst_all[pl.ds(TAILB + nf, 16)]
            sidx[pl.ds(nf, 16)] = jnp.where(lane < TAILN - nf, dv, N)
            for o in range(nf + 16, KC, 16):
                sidx[pl.ds(o, 16)] = jnp.full((16,), N, jnp.int32)

        def zero_acc():
            plsc.subcore_barrier()

            @pl.when(sid < WT)
            def _():
                pltpu.sync_copy(zeros, acc.at[pl.ds(sid * RPT, RPT)])

            plsc.subcore_barrier()

        def writeout(dest, obase):
            plsc.subcore_barrier()

            @pl.when(sid < WT)
            def _():
                pltpu.sync_copy(acc.at[pl.ds(sid * RPT, RPT)],
                                dest.at[pl.ds(obase + sid * RPT, RPT)])

        def startG(tab, c, rows, sem):
            pltpu.async_copy(tab.at[src_all.at[pl.ds(c * KC, KC)]],
                             rows, sem)

        def startS(rows, sidx, sem):
            pltpu.async_copy(rows, acc.at[sidx], sem, add=True)

        def wait(sem):
            pltpu.make_async_copy(tabs[0].at[pl.ds(0, KC)], rows,
                                  sem).wait()

        for s in range(S):
            tab = tabs[s]
            zero_acc()

            def chunk(c, _):
                startG(tab, c, rows, semG)
                build_s(c, sidx)
                wait(semG)
                startS(rows, sidx, semS)
                wait(semS)
                return 0

            lax.fori_loop(0, FULL, chunk, 0)
            startG(tab, CH - 1, rows, semG)
            build_s_tail(sidx)
            wait(semG)
            startS(rows, sidx, semS)
            wait(semS)

            writeout(out, (cid * S + s) * N)

        if with_deg:
            zero_acc()
            pltpu.sync_copy(ones, rows)

            def dchunk(c, _):
                build_s(c, sidx)
                startS(rows, sidx, semS)
                wait(semS)
                return 0

            lax.fori_loop(0, FULL, dchunk, 0)
            build_s_tail(sidx)
            startS(rows, sidx, semS)
            wait(semS)

            writeout(dout, cid * N)

    return spmm


def _inv_deg(dp):
    deg = dp[0, :, 0:1] + dp[1, :, 0:1]
    return jnp.where(deg > 0.0, 1.0 / jnp.maximum(deg, 1.0), 0.0)


def _tc_layer(parts, degp, xins, wa, wx, b, g, bn, stage2=None, *, R=1000,
              interpret=False):
    SX = len(xins)
    N = xins[0].shape[0]
    C = SX * 128
    S = parts.shape[1]
    H = wa.shape[1]

    def body(*refs):
        p_ref, d_ref = refs[0], refs[1]
        x_refs = refs[2:2 + SX]
        wa_ref, wx_ref, b_ref, g_ref, bn_ref = refs[2 + SX:7 + SX]
        rest = refs[7 + SX:]
        p = p_ref[...]
        ps = p[0] + p[1]
        inv = _inv_deg(d_ref[...])
        agg = jnp.concatenate([ps[s] for s in range(S)], axis=-1) * inv
        xcat = jnp.concatenate([x[...] for x in x_refs], axis=-1)
        h = (jnp.dot(agg, wa_ref[...], preferred_element_type=jnp.float32)
             + jnp.dot(xcat, wx_ref[...],
                       preferred_element_type=jnp.float32)
             + b_ref[...])
        mu = jnp.mean(h, axis=-1, keepdims=True)
        var = jnp.mean((h - mu) ** 2, axis=-1, keepdims=True)
        h = (h - mu) * lax.rsqrt(var + EPS) * g_ref[...] + bn_ref[...]
        h = jnp.maximum(h, 0.0)
        if stage2 is None:
            for t in range(H // 128):
                rest[t][...] = h[:, t * 128:(t + 1) * 128]
        else:
            wa2_ref, wx2_ref, b2_ref = rest[:3]
            ya = jnp.dot(h, wa2_ref[...], preferred_element_type=jnp.float32)
            H2 = wa2_ref.shape[1]
            for t in range(H2 // 128):
                rest[3 + t][...] = ya[:, t * 128:(t + 1) * 128]
            rest[3 + H2 // 128][...] = \
                jnp.dot(h, wx2_ref[...],
                        preferred_element_type=jnp.float32) + b2_ref[...]

    full = lambda i: (0, 0)
    row = lambda i: (i, 0)
    slab_spec = pl.BlockSpec((R, 128), row)
    in_specs = [
        pl.BlockSpec((NC, S, R, 128), lambda i: (0, 0, i, 0)),
        pl.BlockSpec((NC, R, 128), lambda i: (0, i, 0)),
    ] + [slab_spec] * SX + [
        pl.BlockSpec((C, H), full),
        pl.BlockSpec((C, H), full),
        pl.BlockSpec((1, H), full),
        pl.BlockSpec((1, H), full),
        pl.BlockSpec((1, H), full),
    ]
    args = [parts, degp] + list(xins) + [wa, wx, b, g, bn]
    slab_t = jax.ShapeDtypeStruct((N, 128), jnp.float32)
    if stage2 is None:
        out_specs = [slab_spec] * (H // 128)
        out_shape = [slab_t] * (H // 128)
    else:
        wa2, wx2, b2 = stage2
        H2 = wa2.shape[1]
        in_specs += [pl.BlockSpec((H, H2), full), pl.BlockSpec((H, H2), full),
                     pl.BlockSpec((1, H2), full)]
        args += [wa2, wx2, b2]
        out_specs = [slab_spec] * (H2 // 128) + [pl.BlockSpec((R, H2), row)]
        out_shape = [slab_t] * (H2 // 128) + \
            [jax.ShapeDtypeStruct((N, H2), jnp.float32)]

    res = pl.pallas_call(
        body,
        grid=(N // R,),
        in_specs=in_specs,
        out_specs=out_specs,
        out_shape=out_shape,
        interpret=interpret,
    )(*args)
    if stage2 is None:
        return res
    return res[:-1], res[-1]


def _tc_final(parts, degp, yx, *, R=1000, interpret=False):
    N, H = yx.shape
    S = H // 128

    def body(p_ref, d_ref, y_ref, o_ref):
        p = p_ref[...]
        ps = p[0] + p[1]
        inv = _inv_deg(d_ref[...])
        agg = jnp.concatenate([ps[s] for s in range(S)], axis=-1) * inv
        o_ref[...] = agg + y_ref[...]

    return pl.pallas_call(
        body,
        grid=(N // R,),
        in_specs=[
            pl.BlockSpec((NC, S, R, 128), lambda i: (0, 0, i, 0)),
            pl.BlockSpec((NC, R, 128), lambda i: (0, i, 0)),
            pl.BlockSpec((R, H), lambda i: (i, 0)),
        ],
        out_specs=pl.BlockSpec((R, H), lambda i: (i, 0)),
        out_shape=jax.ShapeDtypeStruct((N, H), jnp.float32),
        interpret=interpret,
    )(parts, degp, yx)


def kernel(x, edge_index, W0, b0, W1, b1, W2, b2, g0, bn0, g1, bn1):
    N, C0 = x.shape
    E = edge_index.shape[1]
    H = W0.shape[0]
    src = edge_index[0].astype(jnp.int32)
    dst = edge_index[1].astype(jnp.int32)

    Wt0, Wt1, Wt2 = W0.T, W1.T, W2.T
    wa0, wx0 = Wt0[:C0], Wt0[C0:]
    wa1, wx1 = Wt1[:H], Wt1[H:]
    wa2, wx2 = Wt2[:H], Wt2[H:]
    b0r, g0r, bn0r = b0.reshape(1, -1), g0.reshape(1, -1), bn0.reshape(1, -1)
    b1r, g1r, bn1r = b1.reshape(1, -1), g1.reshape(1, -1), bn1.reshape(1, -1)
    b2r = b2.reshape(1, -1)

    zrows = jnp.zeros((N // WT, 128), jnp.float32)
    orows = jnp.ones((KC, 128), jnp.float32)

    S0 = C0 // 128
    x_slabs = [x[:, 128 * s:128 * (s + 1)] for s in range(S0)]

    p0, degp = _make_spmm(N, E, S0, with_deg=True)(
        *x_slabs, src, dst, zrows, orows)
    degp = degp.reshape(NC, N, 128)
    h0_slabs = _tc_layer(p0.reshape(NC, S0, N, 128), degp, x_slabs,
                         wa0, wx0, b0r, g0r, bn0r)

    SH = H // 128
    p1 = _make_spmm(N, E, SH)(*h0_slabs, src, dst, zrows, orows)
    ya_slabs, yx = _tc_layer(p1.reshape(NC, SH, N, 128), degp, h0_slabs,
                             wa1, wx1, b1r, g1r, bn1r,
                             stage2=(wa2, wx2, b2r))

    SO = len(ya_slabs)
    p2 = _make_spmm(N, E, SO)(*ya_slabs, src, dst, zrows, orows)
    out = _tc_final(p2.reshape(NC, SO, N, 128), degp, yx)
    return out


# speedup vs baseline: 2.7418x; 2.7418x over previous
import functools

import jax
import jax.numpy as jnp
from jax import lax
from jax.experimental import pallas as pl
from jax.experimental.pallas import tpu as pltpu
from jax.experimental.pallas import tpu_sc as plsc

EPS = 1e-5

NC = 2
NS = 16
NW = NC * NS

KC = 200
WT = 10


def _make_spmm(N, E, S, with_deg=False, interpret=False):
    EPT = E // NW
    CH = EPT // KC
    assert CH * KC == EPT and KC % 8 == 0
    NI = KC // 16
    OVL = KC - 16
    RPT = N // WT
    mesh = plsc.VectorSubcoreMesh(core_axis_name="c", subcore_axis_name="s")

    parts_t = jax.ShapeDtypeStruct((NC * S * N, 128), jnp.float32)
    out_t = [parts_t, jax.ShapeDtypeStruct((NC * N, 128), jnp.float32)] \
        if with_deg else parts_t

    @functools.partial(
        pl.kernel,
        out_type=out_t,
        mesh=mesh,
        interpret=interpret,
        scratch_types=[
            pltpu.VMEM((EPT,), jnp.int32),
            pltpu.VMEM((EPT,), jnp.int32),
            pltpu.VMEM((KC,), jnp.int32),
            pltpu.VMEM((KC,), jnp.int32),
            pltpu.VMEM((KC, 128), jnp.float32),
            pltpu.VMEM_SHARED((N, 128), jnp.float32),
            pltpu.SemaphoreType.DMA,
        ],
    )
    def spmm(*refs):
        tabs = refs[:S]
        src, dst, zeros, ones = refs[S:S + 4]
        if with_deg:
            (out, dout, src_all, dst_all, gidx, sidx, rows, acc,
             sem) = refs[S + 4:]
        else:
            (out, src_all, dst_all, gidx, sidx, rows, acc,
             sem) = refs[S + 4:]
        cid = lax.axis_index("c")
        sid = lax.axis_index("s")
        wid = cid * NS + sid
        ebase = wid * EPT

        pltpu.sync_copy(src.at[pl.ds(ebase, EPT)], src_all)
        pltpu.sync_copy(dst.at[pl.ds(ebase, EPT)], dst_all)

        def zero_acc():
            plsc.subcore_barrier()

            @pl.when(sid < WT)
            def _():
                pltpu.sync_copy(zeros, acc.at[pl.ds(sid * RPT, RPT)])

            plsc.subcore_barrier()

        def writeout(dest, obase):
            plsc.subcore_barrier()

            @pl.when(sid < WT)
            def _():
                pltpu.sync_copy(acc.at[pl.ds(sid * RPT, RPT)],
                                dest.at[pl.ds(obase + sid * RPT, RPT)])

        for s in range(S):
            tab = tabs[s]
            zero_acc()

            def chunk(g, _):
                for i in range(NI):
                    gidx[pl.ds(i * 16, 16)] = \
                        src_all[pl.ds(g * KC + i * 16, 16)]
                    sidx[pl.ds(i * 16, 16)] = \
                        dst_all[pl.ds(g * KC + i * 16, 16)]
                if KC % 16 != 0:
                    gidx[pl.ds(OVL, 16)] = src_all[pl.ds(g * KC + OVL, 16)]
                    sidx[pl.ds(OVL, 16)] = dst_all[pl.ds(g * KC + OVL, 16)]
                pltpu.async_copy(tab.at[gidx], rows, sem).wait()
                pltpu.sync_copy(rows, acc.at[sidx], add=True)
                return 0

            lax.fori_loop(0, CH, chunk, 0)
            writeout(out, (cid * S + s) * N)

        if with_deg:
            zero_acc()
            pltpu.sync_copy(ones, rows)

            def dchunk(g, _):
                for i in range(NI):
                    sidx[pl.ds(i * 16, 16)] = \
                        dst_all[pl.ds(g * KC + i * 16, 16)]
                if KC % 16 != 0:
                    sidx[pl.ds(OVL, 16)] = dst_all[pl.ds(g * KC + OVL, 16)]
                pltpu.sync_copy(rows, acc.at[sidx], add=True)
                return 0

            lax.fori_loop(0, CH, dchunk, 0)
            writeout(dout, cid * N)

    return spmm


def _inv_deg(dp):
    deg = dp[0, :, 0:1] + dp[1, :, 0:1]
    return jnp.where(deg > 0.0, 1.0 / jnp.maximum(deg, 1.0), 0.0)


def _tc_layer(parts, degp, xins, wa, wx, b, g, bn, stage2=None, *, R=400,
              interpret=False):
    SX = len(xins)
    N = xins[0].shape[0]
    C = SX * 128
    S = parts.shape[1]
    H = wa.shape[1]

    def body(*refs):
        p_ref, d_ref = refs[0], refs[1]
        x_refs = refs[2:2 + SX]
        wa_ref, wx_ref, b_ref, g_ref, bn_ref = refs[2 + SX:7 + SX]
        rest = refs[7 + SX:]
        p = p_ref[...]
        ps = p[0] + p[1]
        inv = _inv_deg(d_ref[...])
        agg = jnp.concatenate([ps[s] for s in range(S)], axis=-1) * inv
        xcat = jnp.concatenate([x[...] for x in x_refs], axis=-1)
        h = (jnp.dot(agg, wa_ref[...], preferred_element_type=jnp.float32)
             + jnp.dot(xcat, wx_ref[...],
                       preferred_element_type=jnp.float32)
             + b_ref[...])
        mu = jnp.mean(h, axis=-1, keepdims=True)
        var = jnp.mean((h - mu) ** 2, axis=-1, keepdims=True)
        h = (h - mu) * lax.rsqrt(var + EPS) * g_ref[...] + bn_ref[...]
        h = jnp.maximum(h, 0.0)
        if stage2 is None:
            for t in range(H // 128):
                rest[t][...] = h[:, t * 128:(t + 1) * 128]
        else:
            wa2_ref, wx2_ref, b2_ref = rest[:3]
            ya = jnp.dot(h, wa2_ref[...], preferred_element_type=jnp.float32)
            H2 = wa2_ref.shape[1]
            for t in range(H2 // 128):
                rest[3 + t][...] = ya[:, t * 128:(t + 1) * 128]
            rest[3 + H2 // 128][...] = \
                jnp.dot(h, wx2_ref[...],
                        preferred_element_type=jnp.float32) + b2_ref[...]

    full = lambda i: (0, 0)
    row = lambda i: (i, 0)
    slab_spec = pl.BlockSpec((R, 128), row)
    in_specs = [
        pl.BlockSpec((NC, S, R, 128), lambda i: (0, 0, i, 0)),
        pl.BlockSpec((NC, R, 128), lambda i: (0, i, 0)),
    ] + [slab_spec] * SX + [
        pl.BlockSpec((C, H), full),
        pl.BlockSpec((C, H), full),
        pl.BlockSpec((1, H), full),
        pl.BlockSpec((1, H), full),
        pl.BlockSpec((1, H), full),
    ]
    args = [parts, degp] + list(xins) + [wa, wx, b, g, bn]
    slab_t = jax.ShapeDtypeStruct((N, 128), jnp.float32)
    if stage2 is None:
        out_specs = [slab_spec] * (H // 128)
        out_shape = [slab_t] * (H // 128)
    else:
        wa2, wx2, b2 = stage2
        H2 = wa2.shape[1]
        in_specs += [pl.BlockSpec((H, H2), full), pl.BlockSpec((H, H2), full),
                     pl.BlockSpec((1, H2), full)]
        args += [wa2, wx2, b2]
        out_specs = [slab_spec] * (H2 // 128) + [pl.BlockSpec((R, H2), row)]
        out_shape = [slab_t] * (H2 // 128) + \
            [jax.ShapeDtypeStruct((N, H2), jnp.float32)]

    res = pl.pallas_call(
        body,
        grid=(N // R,),
        in_specs=in_specs,
        out_specs=out_specs,
        out_shape=out_shape,
        interpret=interpret,
    )(*args)
    if stage2 is None:
        return res
    return res[:-1], res[-1]


def _tc_final(parts, degp, yx, *, R=400, interpret=False):
    N, H = yx.shape
    S = H // 128

    def body(p_ref, d_ref, y_ref, o_ref):
        p = p_ref[...]
        ps = p[0] + p[1]
        inv = _inv_deg(d_ref[...])
        agg = jnp.concatenate([ps[s] for s in range(S)], axis=-1) * inv
        o_ref[...] = agg + y_ref[...]

    return pl.pallas_call(
        body,
        grid=(N // R,),
        in_specs=[
            pl.BlockSpec((NC, S, R, 128), lambda i: (0, 0, i, 0)),
            pl.BlockSpec((NC, R, 128), lambda i: (0, i, 0)),
            pl.BlockSpec((R, H), lambda i: (i, 0)),
        ],
        out_specs=pl.BlockSpec((R, H), lambda i: (i, 0)),
        out_shape=jax.ShapeDtypeStruct((N, H), jnp.float32),
        interpret=interpret,
    )(parts, degp, yx)


def kernel(x, edge_index, W0, b0, W1, b1, W2, b2, g0, bn0, g1, bn1):
    N, C0 = x.shape
    E = edge_index.shape[1]
    H = W0.shape[0]
    src = edge_index[0].astype(jnp.int32)
    dst = edge_index[1].astype(jnp.int32)

    Wt0, Wt1, Wt2 = W0.T, W1.T, W2.T
    wa0, wx0 = Wt0[:C0], Wt0[C0:]
    wa1, wx1 = Wt1[:H], Wt1[H:]
    wa2, wx2 = Wt2[:H], Wt2[H:]
    b0r, g0r, bn0r = b0.reshape(1, -1), g0.reshape(1, -1), bn0.reshape(1, -1)
    b1r, g1r, bn1r = b1.reshape(1, -1), g1.reshape(1, -1), bn1.reshape(1, -1)
    b2r = b2.reshape(1, -1)

    zrows = jnp.zeros((N // WT, 128), jnp.float32)
    orows = jnp.ones((KC, 128), jnp.float32)

    S0 = C0 // 128
    x_slabs = [x[:, 128 * s:128 * (s + 1)] for s in range(S0)]

    p0, degp = _make_spmm(N, E, S0, with_deg=True)(
        *x_slabs, src, dst, zrows, orows)
    degp = degp.reshape(NC, N, 128)
    h0_slabs = _tc_layer(p0.reshape(NC, S0, N, 128), degp, x_slabs,
                         wa0, wx0, b0r, g0r, bn0r)

    SH = H // 128
    p1 = _make_spmm(N, E, SH)(*h0_slabs, src, dst, zrows, orows)
    ya_slabs, yx = _tc_layer(p1.reshape(NC, SH, N, 128), degp, h0_slabs,
                             wa1, wx1, b1r, g1r, bn1r,
                             stage2=(wa2, wx2, b2r))

    SO = len(ya_slabs)
    p2 = _make_spmm(N, E, SO)(*ya_slabs, src, dst, zrows, orows)
    out = _tc_final(p2.reshape(NC, SO, N, 128), degp, yx)
    return out


# speedup vs baseline: 2.8275x; 1.0313x over previous
import functools

import jax
import jax.numpy as jnp
from jax import lax
from jax.experimental import pallas as pl
from jax.experimental.pallas import tpu as pltpu
from jax.experimental.pallas import tpu_sc as plsc

EPS = 1e-5

NC = 2
NS = 16
NW = NC * NS

KC = 200
WT = 10


def _make_spmm(N, E, S, with_deg=False, interpret=False):
    EPT = E // NW
    CH = EPT // KC
    assert CH * KC == EPT and KC % 8 == 0
    NI = KC // 16
    OVL = KC - 16
    RPT = N // WT
    mesh = plsc.VectorSubcoreMesh(core_axis_name="c", subcore_axis_name="s")

    parts_t = jax.ShapeDtypeStruct((NC * S * N, 128), jnp.float32)
    out_t = [parts_t, jax.ShapeDtypeStruct((NC * N, 128), jnp.float32)] \
        if with_deg else parts_t

    @functools.partial(
        pl.kernel,
        out_type=out_t,
        mesh=mesh,
        interpret=interpret,
        scratch_types=[
            pltpu.VMEM((EPT,), jnp.int32),
            pltpu.VMEM((EPT,), jnp.int32),
            pltpu.VMEM((KC,), jnp.int32),
            pltpu.VMEM((KC,), jnp.int32),
            pltpu.VMEM((KC, 128), jnp.float32),
            pltpu.VMEM_SHARED((N, 128), jnp.float32),
            pltpu.SemaphoreType.DMA,
        ],
    )
    def spmm(*refs):
        tabs = refs[:S]
        src, dst, zeros, ones = refs[S:S + 4]
        if with_deg:
            (out, dout, src_all, dst_all, gidx, sidx, rows, acc,
             sem) = refs[S + 4:]
        else:
            (out, src_all, dst_all, gidx, sidx, rows, acc,
             sem) = refs[S + 4:]
        cid = lax.axis_index("c")
        sid = lax.axis_index("s")
        wid = cid * NS + sid
        ebase = wid * EPT

        pltpu.sync_copy(src.at[pl.ds(ebase, EPT)], src_all)
        pltpu.sync_copy(dst.at[pl.ds(ebase, EPT)], dst_all)

        def zero_acc():
            plsc.subcore_barrier()

            @pl.when(sid < WT)
            def _():
                pltpu.sync_copy(zeros, acc.at[pl.ds(sid * RPT, RPT)])

            plsc.subcore_barrier()

        def writeout(dest, obase):
            plsc.subcore_barrier()

            @pl.when(sid < WT)
            def _():
                pltpu.sync_copy(acc.at[pl.ds(sid * RPT, RPT)],
                                dest.at[pl.ds(obase + sid * RPT, RPT)])

        for s in range(S):
            tab = tabs[s]
            zero_acc()

            def chunk(g, _):
                for i in range(NI):
                    gidx[pl.ds(i * 16, 16)] = \
                        src_all[pl.ds(g * KC + i * 16, 16)]
                    sidx[pl.ds(i * 16, 16)] = \
                        dst_all[pl.ds(g * KC + i * 16, 16)]
                if KC % 16 != 0:
                    gidx[pl.ds(OVL, 16)] = src_all[pl.ds(g * KC + OVL, 16)]
                    sidx[pl.ds(OVL, 16)] = dst_all[pl.ds(g * KC + OVL, 16)]
                pltpu.async_copy(tab.at[gidx], rows, sem).wait()
                pltpu.sync_copy(rows, acc.at[sidx], add=True)
                return 0

            lax.fori_loop(0, CH, chunk, 0)
            writeout(out, (cid * S + s) * N)

        if with_deg:
            zero_acc()
            pltpu.sync_copy(ones, rows)

            def dchunk(g, _):
                for i in range(NI):
                    sidx[pl.ds(i * 16, 16)] = \
                        dst_all[pl.ds(g * KC + i * 16, 16)]
                if KC % 16 != 0:
                    sidx[pl.ds(OVL, 16)] = dst_all[pl.ds(g * KC + OVL, 16)]
                pltpu.sync_copy(rows, acc.at[sidx], add=True)
                return 0

            lax.fori_loop(0, CH, dchunk, 0)
            writeout(dout, cid * N)

    return spmm


def _inv_deg(dp):
    deg = dp[0, :, 0:1] + dp[1, :, 0:1]
    return jnp.where(deg > 0.0, 1.0 / jnp.maximum(deg, 1.0), 0.0)


def _tc_layer(parts, degp, xins, wa, wx, b, g, bn, stage2=None, *, R=1000,
              interpret=False):
    SX = len(xins)
    N = xins[0].shape[0]
    C = SX * 128
    S = parts.shape[1]
    H = wa.shape[1]

    def body(*refs):
        p_ref, d_ref = refs[0], refs[1]
        x_refs = refs[2:2 + SX]
        wa_ref, wx_ref, b_ref, g_ref, bn_ref = refs[2 + SX:7 + SX]
        rest = refs[7 + SX:]
        p = p_ref[...]
        ps = p[0] + p[1]
        inv = _inv_deg(d_ref[...])
        agg = jnp.concatenate([ps[s] for s in range(S)], axis=-1) * inv
        xcat = jnp.concatenate([x[...] for x in x_refs], axis=-1)
        h = (jnp.dot(agg, wa_ref[...], preferred_element_type=jnp.float32)
             + jnp.dot(xcat, wx_ref[...],
                       preferred_element_type=jnp.float32)
             + b_ref[...])
        mu = jnp.mean(h, axis=-1, keepdims=True)
        var = jnp.mean((h - mu) ** 2, axis=-1, keepdims=True)
        h = (h - mu) * lax.rsqrt(var + EPS) * g_ref[...] + bn_ref[...]
        h = jnp.maximum(h, 0.0)
        if stage2 is None:
            for t in range(H // 128):
                rest[t][...] = h[:, t * 128:(t + 1) * 128]
        else:
            wa2_ref, wx2_ref, b2_ref = rest[:3]
            ya = jnp.dot(h, wa2_ref[...], preferred_element_type=jnp.float32)
            H2 = wa2_ref.shape[1]
            for t in range(H2 // 128):
                rest[3 + t][...] = ya[:, t * 128:(t + 1) * 128]
            rest[3 + H2 // 128][...] = \
                jnp.dot(h, wx2_ref[...],
                        preferred_element_type=jnp.float32) + b2_ref[...]

    full = lambda i: (0, 0)
    row = lambda i: (i, 0)
    slab_spec = pl.BlockSpec((R, 128), row)
    in_specs = [
        pl.BlockSpec((NC, S, R, 128), lambda i: (0, 0, i, 0)),
        pl.BlockSpec((NC, R, 128), lambda i: (0, i, 0)),
    ] + [slab_spec] * SX + [
        pl.BlockSpec((C, H), full),
        pl.BlockSpec((C, H), full),
        pl.BlockSpec((1, H), full),
        pl.BlockSpec((1, H), full),
        pl.BlockSpec((1, H), full),
    ]
    args = [parts, degp] + list(xins) + [wa, wx, b, g, bn]
    slab_t = jax.ShapeDtypeStruct((N, 128), jnp.float32)
    if stage2 is None:
        out_specs = [slab_spec] * (H // 128)
        out_shape = [slab_t] * (H // 128)
    else:
        wa2, wx2, b2 = stage2
        H2 = wa2.shape[1]
        in_specs += [pl.BlockSpec((H, H2), full), pl.BlockSpec((H, H2), full),
                     pl.BlockSpec((1, H2), full)]
        args += [wa2, wx2, b2]
        out_specs = [slab_spec] * (H2 // 128) + [pl.BlockSpec((R, H2), row)]
        out_shape = [slab_t] * (H2 // 128) + \
            [jax.ShapeDtypeStruct((N, H2), jnp.float32)]

    res = pl.pallas_call(
        body,
        grid=(N // R,),
        in_specs=in_specs,
        out_specs=out_specs,
        out_shape=out_shape,
        interpret=interpret,
    )(*args)
    if stage2 is None:
        return res
    return res[:-1], res[-1]


def _tc_final(parts, degp, yx, *, R=1000, interpret=False):
    N, H = yx.shape
    S = H // 128

    def body(p_ref, d_ref, y_ref, o_ref):
        p = p_ref[...]
        ps = p[0] + p[1]
        inv = _inv_deg(d_ref[...])
        agg = jnp.concatenate([ps[s] for s in range(S)], axis=-1) * inv
        o_ref[...] = agg + y_ref[...]

    return pl.pallas_call(
        body,
        grid=(N // R,),
        in_specs=[
            pl.BlockSpec((NC, S, R, 128), lambda i: (0, 0, i, 0)),
            pl.BlockSpec((NC, R, 128), lambda i: (0, i, 0)),
            pl.BlockSpec((R, H), lambda i: (i, 0)),
        ],
        out_specs=pl.BlockSpec((R, H), lambda i: (i, 0)),
        out_shape=jax.ShapeDtypeStruct((N, H), jnp.float32),
        interpret=interpret,
    )(parts, degp, yx)


def kernel(x, edge_index, W0, b0, W1, b1, W2, b2, g0, bn0, g1, bn1):
    N, C0 = x.shape
    E = edge_index.shape[1]
    H = W0.shape[0]
    src = edge_index[0].astype(jnp.int32)
    dst = edge_index[1].astype(jnp.int32)

    Wt0, Wt1, Wt2 = W0.T, W1.T, W2.T
    wa0, wx0 = Wt0[:C0], Wt0[C0:]
    wa1, wx1 = Wt1[:H], Wt1[H:]
    wa2, wx2 = Wt2[:H], Wt2[H:]
    b0r, g0r, bn0r = b0.reshape(1, -1), g0.reshape(1, -1), bn0.reshape(1, -1)
    b1r, g1r, bn1r = b1.reshape(1, -1), g1.reshape(1, -1), bn1.reshape(1, -1)
    b2r = b2.reshape(1, -1)

    zrows = jnp.zeros((N // WT, 128), jnp.float32)
    orows = jnp.ones((KC, 128), jnp.float32)

    S0 = C0 // 128
    x_slabs = [x[:, 128 * s:128 * (s + 1)] for s in range(S0)]

    p0, degp = _make_spmm(N, E, S0, with_deg=True)(
        *x_slabs, src, dst, zrows, orows)
    degp = degp.reshape(NC, N, 128)
    h0_slabs = _tc_layer(p0.reshape(NC, S0, N, 128), degp, x_slabs,
                         wa0, wx0, b0r, g0r, bn0r)

    SH = H // 128
    p1 = _make_spmm(N, E, SH)(*h0_slabs, src, dst, zrows, orows)
    ya_slabs, yx = _tc_layer(p1.reshape(NC, SH, N, 128), degp, h0_slabs,
                             wa1, wx1, b1r, g1r, bn1r,
                             stage2=(wa2, wx2, b2r))

    SO = len(ya_slabs)
    p2 = _make_spmm(N, E, SO)(*ya_slabs, src, dst, zrows, orows)
    out = _tc_final(p2.reshape(NC, SO, N, 128), degp, yx)
    return out


# speedup vs baseline: 2.8871x; 1.0211x over previous
import functools

import jax
import jax.numpy as jnp
from jax import lax
from jax.experimental import pallas as pl
from jax.experimental.pallas import tpu as pltpu
from jax.experimental.pallas import tpu_sc as plsc

EPS = 1e-5

NC = 2
NS = 16
NW = NC * NS

KC = 200
WT = 10


def _make_spmm(N, E, S, with_deg=False, interpret=False):
    EPT = E // NW
    CH = EPT // KC
    assert CH * KC == EPT and KC % 8 == 0
    NI = KC // 16
    OVL = KC - 16
    RPT = N // WT
    mesh = plsc.VectorSubcoreMesh(core_axis_name="c", subcore_axis_name="s")

    parts_t = jax.ShapeDtypeStruct((NC * S * N, 128), jnp.float32)
    out_t = [parts_t, jax.ShapeDtypeStruct((NC * N, 128), jnp.float32)] \
        if with_deg else parts_t

    @functools.partial(
        pl.kernel,
        out_type=out_t,
        mesh=mesh,
        interpret=interpret,
        scratch_types=[
            pltpu.VMEM((EPT,), jnp.int32),
            pltpu.VMEM((EPT,), jnp.int32),
            pltpu.VMEM((KC,), jnp.int32),
            pltpu.VMEM((KC,), jnp.int32),
            pltpu.VMEM((KC,), jnp.int32),
            pltpu.VMEM((KC,), jnp.int32),
            pltpu.VMEM((KC, 128), jnp.float32),
            pltpu.VMEM_SHARED((N, 128), jnp.float32),
            pltpu.SemaphoreType.DMA,
        ],
    )
    def spmm(*refs):
        tabs = refs[:S]
        src, dst, zeros, ones = refs[S:S + 4]
        if with_deg:
            (out, dout, src_all, dst_all, gidxA, sidxA, gidxB, sidxB,
             rows, acc, sem) = refs[S + 4:]
        else:
            (out, src_all, dst_all, gidxA, sidxA, gidxB, sidxB,
             rows, acc, sem) = refs[S + 4:]
        cid = lax.axis_index("c")
        sid = lax.axis_index("s")
        wid = cid * NS + sid
        ebase = wid * EPT

        pltpu.sync_copy(src.at[pl.ds(ebase, EPT)], src_all)
        pltpu.sync_copy(dst.at[pl.ds(ebase, EPT)], dst_all)

        def zero_acc():
            plsc.subcore_barrier()

            @pl.when(sid < WT)
            def _():
                pltpu.sync_copy(zeros, acc.at[pl.ds(sid * RPT, RPT)])

            plsc.subcore_barrier()

        def writeout(dest, obase):
            plsc.subcore_barrier()

            @pl.when(sid < WT)
            def _():
                pltpu.sync_copy(acc.at[pl.ds(sid * RPT, RPT)],
                                dest.at[pl.ds(obase + sid * RPT, RPT)])

        def build(g, gidx, sidx):
            for i in range(NI):
                gidx[pl.ds(i * 16, 16)] = \
                    src_all[pl.ds(g * KC + i * 16, 16)]
                sidx[pl.ds(i * 16, 16)] = \
                    dst_all[pl.ds(g * KC + i * 16, 16)]
            if KC % 16 != 0:
                gidx[pl.ds(OVL, 16)] = src_all[pl.ds(g * KC + OVL, 16)]
                sidx[pl.ds(OVL, 16)] = dst_all[pl.ds(g * KC + OVL, 16)]

        for s in range(S):
            tab = tabs[s]
            zero_acc()
            build(0, gidxA, sidxA)

            def pairc(i, _):
                g = 2 * i
                pltpu.async_copy(tab.at[gidxA], rows, sem)
                build(g + 1, gidxB, sidxB)
                pltpu.make_async_copy(tab.at[gidxA], rows, sem).wait()
                pltpu.sync_copy(rows, acc.at[sidxA], add=True)
                pltpu.async_copy(tab.at[gidxB], rows, sem)
                build(g + 2, gidxA, sidxA)
                pltpu.make_async_copy(tab.at[gidxB], rows, sem).wait()
                pltpu.sync_copy(rows, acc.at[sidxB], add=True)
                return 0

            lax.fori_loop(0, (CH - 1) // 2, pairc, 0)
            pltpu.async_copy(tab.at[gidxA], rows, sem).wait()
            pltpu.sync_copy(rows, acc.at[sidxA], add=True)
            writeout(out, (cid * S + s) * N)

        if with_deg:
            zero_acc()
            pltpu.sync_copy(ones, rows)

            def dchunk(g, _):
                for i in range(NI):
                    sidxA[pl.ds(i * 16, 16)] = \
                        dst_all[pl.ds(g * KC + i * 16, 16)]
                if KC % 16 != 0:
                    sidxA[pl.ds(OVL, 16)] = dst_all[pl.ds(g * KC + OVL, 16)]
                pltpu.sync_copy(rows, acc.at[sidxA], add=True)
                return 0

            lax.fori_loop(0, CH, dchunk, 0)
            writeout(dout, cid * N)

    return spmm


def _inv_deg(dp):
    deg = dp[0, :, 0:1] + dp[1, :, 0:1]
    return jnp.where(deg > 0.0, 1.0 / jnp.maximum(deg, 1.0), 0.0)


def _tc_layer(parts, degp, xins, wa, wx, b, g, bn, stage2=None, *, R=1000,
              interpret=False):
    SX = len(xins)
    N = xins[0].shape[0]
    C = SX * 128
    S = parts.shape[1]
    H = wa.shape[1]

    def body(*refs):
        p_ref, d_ref = refs[0], refs[1]
        x_refs = refs[2:2 + SX]
        wa_ref, wx_ref, b_ref, g_ref, bn_ref = refs[2 + SX:7 + SX]
        rest = refs[7 + SX:]
        p = p_ref[...]
        ps = p[0] + p[1]
        inv = _inv_deg(d_ref[...])
        agg = jnp.concatenate([ps[s] for s in range(S)], axis=-1) * inv
        xcat = jnp.concatenate([x[...] for x in x_refs], axis=-1)
        h = (jnp.dot(agg, wa_ref[...], preferred_element_type=jnp.float32)
             + jnp.dot(xcat, wx_ref[...],
                       preferred_element_type=jnp.float32)
             + b_ref[...])
        mu = jnp.mean(h, axis=-1, keepdims=True)
        var = jnp.mean((h - mu) ** 2, axis=-1, keepdims=True)
        h = (h - mu) * lax.rsqrt(var + EPS) * g_ref[...] + bn_ref[...]
        h = jnp.maximum(h, 0.0)
        if stage2 is None:
            for t in range(H // 128):
                rest[t][...] = h[:, t * 128:(t + 1) * 128]
        else:
            wa2_ref, wx2_ref, b2_ref = rest[:3]
            ya = jnp.dot(h, wa2_ref[...], preferred_element_type=jnp.float32)
            H2 = wa2_ref.shape[1]
            for t in range(H2 // 128):
                rest[3 + t][...] = ya[:, t * 128:(t + 1) * 128]
            rest[3 + H2 // 128][...] = \
                jnp.dot(h, wx2_ref[...],
                        preferred_element_type=jnp.float32) + b2_ref[...]

    full = lambda i: (0, 0)
    row = lambda i: (i, 0)
    slab_spec = pl.BlockSpec((R, 128), row)
    in_specs = [
        pl.BlockSpec((NC, S, R, 128), lambda i: (0, 0, i, 0)),
        pl.BlockSpec((NC, R, 128), lambda i: (0, i, 0)),
    ] + [slab_spec] * SX + [
        pl.BlockSpec((C, H), full),
        pl.BlockSpec((C, H), full),
        pl.BlockSpec((1, H), full),
        pl.BlockSpec((1, H), full),
        pl.BlockSpec((1, H), full),
    ]
    args = [parts, degp] + list(xins) + [wa, wx, b, g, bn]
    slab_t = jax.ShapeDtypeStruct((N, 128), jnp.float32)
    if stage2 is None:
        out_specs = [slab_spec] * (H // 128)
        out_shape = [slab_t] * (H // 128)
    else:
        wa2, wx2, b2 = stage2
        H2 = wa2.shape[1]
        in_specs += [pl.BlockSpec((H, H2), full), pl.BlockSpec((H, H2), full),
                     pl.BlockSpec((1, H2), full)]
        args += [wa2, wx2, b2]
        out_specs = [slab_spec] * (H2 // 128) + [pl.BlockSpec((R, H2), row)]
        out_shape = [slab_t] * (H2 // 128) + \
            [jax.ShapeDtypeStruct((N, H2), jnp.float32)]

    res = pl.pallas_call(
        body,
        grid=(N // R,),
        in_specs=in_specs,
        out_specs=out_specs,
        out_shape=out_shape,
        interpret=interpret,
    )(*args)
    if stage2 is None:
        return res
    return res[:-1], res[-1]


def _tc_final(parts, degp, yx, *, R=1000, interpret=False):
    N, H = yx.shape
    S = H // 128

    def body(p_ref, d_ref, y_ref, o_ref):
        p = p_ref[...]
        ps = p[0] + p[1]
        inv = _inv_deg(d_ref[...])
        agg = jnp.concatenate([ps[s] for s in range(S)], axis=-1) * inv
        o_ref[...] = agg + y_ref[...]

    return pl.pallas_call(
        body,
        grid=(N // R,),
        in_specs=[
            pl.BlockSpec((NC, S, R, 128), lambda i: (0, 0, i, 0)),
            pl.BlockSpec((NC, R, 128), lambda i: (0, i, 0)),
            pl.BlockSpec((R, H), lambda i: (i, 0)),
        ],
        out_specs=pl.BlockSpec((R, H), lambda i: (i, 0)),
        out_shape=jax.ShapeDtypeStruct((N, H), jnp.float32),
        interpret=interpret,
    )(parts, degp, yx)


def kernel(x, edge_index, W0, b0, W1, b1, W2, b2, g0, bn0, g1, bn1):
    N, C0 = x.shape
    E = edge_index.shape[1]
    H = W0.shape[0]
    src = edge_index[0].astype(jnp.int32)
    dst = edge_index[1].astype(jnp.int32)

    Wt0, Wt1, Wt2 = W0.T, W1.T, W2.T
    wa0, wx0 = Wt0[:C0], Wt0[C0:]
    wa1, wx1 = Wt1[:H], Wt1[H:]
    wa2, wx2 = Wt2[:H], Wt2[H:]
    b0r, g0r, bn0r = b0.reshape(1, -1), g0.reshape(1, -1), bn0.reshape(1, -1)
    b1r, g1r, bn1r = b1.reshape(1, -1), g1.reshape(1, -1), bn1.reshape(1, -1)
    b2r = b2.reshape(1, -1)

    zrows = jnp.zeros((N // WT, 128), jnp.float32)
    orows = jnp.ones((KC, 128), jnp.float32)

    S0 = C0 // 128
    x_slabs = [x[:, 128 * s:128 * (s + 1)] for s in range(S0)]

    p0, degp = _make_spmm(N, E, S0, with_deg=True)(
        *x_slabs, src, dst, zrows, orows)
    degp = degp.reshape(NC, N, 128)
    h0_slabs = _tc_layer(p0.reshape(NC, S0, N, 128), degp, x_slabs,
                         wa0, wx0, b0r, g0r, bn0r)

    SH = H // 128
    p1 = _make_spmm(N, E, SH)(*h0_slabs, src, dst, zrows, orows)
    ya_slabs, yx = _tc_layer(p1.reshape(NC, SH, N, 128), degp, h0_slabs,
                             wa1, wx1, b1r, g1r, bn1r,
                             stage2=(wa2, wx2, b2r))

    SO = len(ya_slabs)
    p2 = _make_spmm(N, E, SO)(*ya_slabs, src, dst, zrows, orows)
    out = _tc_final(p2.reshape(NC, SO, N, 128), degp, yx)
    return out
